# baseline (speedup 1.0000x reference)
import functools
import math

import jax
import jax.numpy as jnp
from jax import lax
from jax.experimental import pallas as pl
from jax.experimental.pallas import tpu as pltpu
from jax.experimental.pallas import tpu_sc as plsc

D_MODEL = 64
SCALE = math.sqrt(D_MODEL)

NC, NS, L = 2, 16, 16
NW = NC * NS

CHUNK = 128
NBUF = 2


@functools.lru_cache(maxsize=None)
def _make_kernel(n_chunks: int, D: int):
    assert n_chunks % NBUF == 0 and n_chunks // NBUF >= 2
    mesh = plsc.VectorSubcoreMesh(core_axis_name="c", subcore_axis_name="s")

    @functools.partial(
        pl.kernel,
        mesh=mesh,
        out_type=jax.ShapeDtypeStruct((NW, n_chunks, CHUNK, D), jnp.float32),
        scratch_types=[
            pltpu.VMEM((n_chunks, CHUNK), jnp.int32),
            pltpu.VMEM((NBUF, CHUNK, D), jnp.float32),
            pltpu.VMEM((NBUF, CHUNK, D), jnp.float32),
            pltpu.SemaphoreType.DMA,
            pltpu.SemaphoreType.DMA,
            pltpu.SemaphoreType.DMA,
            pltpu.SemaphoreType.DMA,
        ],
        compiler_params=pltpu.CompilerParams(use_tc_tiling_on_sc=False),
    )
    def k(idx_hbm, table_hbm, out_hbm, idx_v, gbuf, sbuf,
          gsem0, gsem1, ssem0, ssem1):
        gsem = (gsem0, gsem1)
        ssem = (ssem0, ssem1)
        wid = lax.axis_index("s") * NC + lax.axis_index("c")
        pltpu.sync_copy(idx_hbm.at[wid], idx_v)

        def gather(j, b):
            pltpu.async_copy(table_hbm.at[idx_v.at[j]], gbuf.at[b], gsem[b])

        def gather_wait(b):
            pltpu.make_async_copy(
                table_hbm.at[idx_v.at[0]], gbuf.at[b], gsem[b]).wait()

        def store(j, b):
            pltpu.async_copy(sbuf.at[b], out_hbm.at[wid, j], ssem[b])

        def store_wait(b):
            pltpu.make_async_copy(
                sbuf.at[b], out_hbm.at[wid, 0], ssem[b]).wait()

        def scale(b):
            def row_body(r, c):
                for cc in range(D // L):
                    sl = pl.ds(cc * L, L)
                    sbuf[b, r, sl] = gbuf[b, r, sl] * SCALE
                return c
            lax.fori_loop(0, CHUNK, row_body, 0, unroll=8)

        for b in range(NBUF):
            gather(b, b)

        for b in range(NBUF):
            gather_wait(b)
            scale(b)
            gather(b + NBUF, b)
            store(b, b)

        def body(g, carry):
            for b in range(NBUF):
                j = g * NBUF + b
                gather_wait(b)
                store_wait(b)
                scale(b)
                gather(j + NBUF, b)
                store(j, b)
            return carry

        lax.fori_loop(1, n_chunks // NBUF - 1, body, 0)

        for b in range(NBUF):
            j = n_chunks - NBUF + b
            gather_wait(b)
            store_wait(b)
            scale(b)
            store(j, b)

        for b in range(NBUF):
            store_wait(b)

    return k


def kernel(x, table):
    B = x.size
    D = table.shape[1]
    n_chunks = B // (NW * CHUNK)
    idx = jnp.reshape(x.astype(jnp.int32), (NW, n_chunks, CHUNK))
    out = _make_kernel(n_chunks, D)(idx, table)
    return jnp.reshape(out, x.shape + (D,))
